# baseline (speedup 1.0000x reference)
import functools

import jax
import jax.numpy as jnp
from jax import lax
from jax.experimental import pallas as pl
from jax.experimental.pallas import tpu as pltpu
from jax.experimental.pallas import tpu_sc as plsc

_NC = 2
_NS = 16
_C = 128


def _enc_body(uf_ref, wb_ref, b_ref, w1_ref, h_ref):
    u = jnp.dot(uf_ref[...], wb_ref[...], preferred_element_type=jnp.float32)
    u = u + b_ref[...]
    u = jnp.where(u > 0, u, 0.01 * u)
    h_ref[...] = jnp.dot(u, w1_ref[...], preferred_element_type=jnp.float32)


def _encoder(uf, wb, b, w1, bn=1000, interpret=False):
    n, f = uf.shape
    d = wb.shape[1]
    return pl.pallas_call(
        _enc_body,
        grid=(n // bn,),
        in_specs=[
            pl.BlockSpec((bn, f), lambda i: (i, 0)),
            pl.BlockSpec((f, d), lambda i: (0, 0)),
            pl.BlockSpec((1, d), lambda i: (0, 0)),
            pl.BlockSpec((d, d), lambda i: (0, 0)),
        ],
        out_specs=pl.BlockSpec((bn, d), lambda i: (i, 0)),
        out_shape=jax.ShapeDtypeStruct((n, d), jnp.float32),
        interpret=interpret,
    )(uf, wb, b, w1)


def _mid_body(p_ref, w2_ref, g_ref):
    z = p_ref[0] + p_ref[1]
    g_ref[...] = jnp.dot(z, w2_ref[...], preferred_element_type=jnp.float32)


def _mid(p, w2, bn=1000, interpret=False):
    _, n, d = p.shape
    o = w2.shape[1]
    return pl.pallas_call(
        _mid_body,
        grid=(n // bn,),
        in_specs=[
            pl.BlockSpec((2, bn, d), lambda i: (0, i, 0)),
            pl.BlockSpec((d, o), lambda i: (0, 0)),
        ],
        out_specs=pl.BlockSpec((bn, o), lambda i: (i, 0)),
        out_shape=jax.ShapeDtypeStruct((n, o), jnp.float32),
        interpret=interpret,
    )(p, w2)


def _zc_body(q_ref, gt_ref, z_ref, c_ref):
    zf = q_ref[0] + q_ref[1]
    z_ref[...] = zf[:, :z_ref.shape[1]]
    zn = jnp.sum(zf * zf, axis=1, keepdims=True)
    gt = gt_ref[...]
    gn = jnp.sum(gt * gt, axis=0, keepdims=True)
    cross = jnp.dot(zf, gt, preferred_element_type=jnp.float32)
    c_ref[...] = zn + gn - 2.0 * cross


def _zclusters(q, gt, out_d, bn=1000, interpret=False):
    _, n, dp = q.shape
    k = gt.shape[1]
    return pl.pallas_call(
        _zc_body,
        grid=(n // bn,),
        in_specs=[
            pl.BlockSpec((2, bn, dp), lambda i: (0, i, 0)),
            pl.BlockSpec((dp, k), lambda i: (0, 0)),
        ],
        out_specs=[
            pl.BlockSpec((bn, out_d), lambda i: (i, 0)),
            pl.BlockSpec((bn, k), lambda i: (i, 0)),
        ],
        out_shape=[
            jax.ShapeDtypeStruct((n, out_d), jnp.float32),
            jax.ShapeDtypeStruct((n, k), jnp.float32),
        ],
        interpret=interpret,
    )(q, gt)


def _recon_body(zi_ref, zj_ref, r_ref):
    r_ref[...] = lax.dot_general(
        zi_ref[...], zj_ref[...], (((1,), (1,)), ((), ())),
        preferred_element_type=jnp.float32)


def _recon(z, bn=400, interpret=False):
    n, d = z.shape
    return pl.pallas_call(
        _recon_body,
        grid=(n // bn,),
        in_specs=[
            pl.BlockSpec((bn, d), lambda i: (i, 0)),
            pl.BlockSpec((n, d), lambda i: (0, 0)),
        ],
        out_specs=pl.BlockSpec((bn, n), lambda i: (i, 0)),
        out_shape=jax.ShapeDtypeStruct((n, n), jnp.float32),
        interpret=interpret,
    )(z, z)


def _make_spmm(n_nodes, d, n_chunk):
    groups = d // 16
    r1 = (n_nodes // _NS) & ~7
    tail = n_nodes - r1 * _NS
    mesh = plsc.VectorSubcoreMesh(core_axis_name="c", subcore_axis_name="s")
    dnums = lax.GatherDimensionNumbers(
        offset_dims=(), collapsed_slice_dims=(0,), start_index_map=(0,))

    @functools.partial(
        pl.kernel, mesh=mesh,
        out_type=jax.ShapeDtypeStruct((_NC, n_nodes, d), jnp.float32),
        scratch_types=[
            pltpu.VMEM((2, _C), jnp.int32),
            pltpu.VMEM((2, _C), jnp.int32),
            pltpu.VMEM((_C,), jnp.float32),
            pltpu.VMEM((_C,), jnp.float32),
            pltpu.VMEM((_C, d), jnp.float32),
            pltpu.VMEM((_C, d), jnp.float32),
            pltpu.VMEM_SHARED((n_nodes, d), jnp.float32),
            pltpu.SemaphoreType.DMA,
            pltpu.SemaphoreType.DMA,
            pltpu.SemaphoreType.DMA,
            pltpu.SemaphoreType.DMA,
            pltpu.SemaphoreType.DMA,
            pltpu.SemaphoreType.DMA,
        ],
    )
    def spmm(x_hbm, edata_hbm, vdata_hbm, zeros_hbm, out_hbm,
             eb0, eb1, vb0, vb1, rb0, rb1, acc, gs0, gs1, ss0, ss1, vs0, vs1):
        ebuf, rbuf, gsem, ssem = (eb0, eb1), (rb0, rb1), (gs0, gs1), (ss0, ss1)
        vbuf, vsem = (vb0, vb1), (vs0, vs1)
        cid = lax.axis_index("c")
        sid = lax.axis_index("s")
        w = cid * _NS + sid
        r0 = sid * r1
        pltpu.sync_copy(zeros_hbm.at[pl.ds(0, r1)], acc.at[pl.ds(r0, r1)])

        @pl.when(sid == 0)
        def _zero_tail():
            pltpu.sync_copy(zeros_hbm.at[pl.ds(0, tail)],
                            acc.at[pl.ds(r1 * _NS, tail)])

        plsc.subcore_barrier()

        def scale(b, _unused):
            def scale_body(t, c2):
                vs = vbuf[b][pl.ds(t * 16, 16)]
                for e in range(16):
                    vv = lax.gather(
                        vs, jnp.full((16, 1), e, jnp.int32), dnums, (1,),
                        mode=lax.GatherScatterMode.PROMISE_IN_BOUNDS)
                    r = t * 16 + e
                    for g in range(groups):
                        sl = pl.ds(g * 16, 16)
                        rbuf[b][r, sl] = rbuf[b][r, sl] * vv
                return c2

            lax.fori_loop(0, _C // 16, scale_body, 0)

        pltpu.sync_copy(edata_hbm.at[w, 0], ebuf[0])
        pltpu.make_async_copy(x_hbm.at[ebuf[0].at[1]], rbuf[0], gsem[0]).start()
        pltpu.make_async_copy(vdata_hbm.at[w, 0], vbuf[0], vsem[0]).start()

        def pair_body(t, carry):
            for b in range(2):
                k = 2 * t + b
                bn = 1 - b

                @pl.when(k >= 1)
                def _wait_prev_scatter():
                    pltpu.make_async_copy(
                        rbuf[bn], acc.at[ebuf[bn].at[0]], ssem[bn]).wait()

                pltpu.sync_copy(edata_hbm.at[w, k + 1], ebuf[bn])
                pltpu.make_async_copy(
                    x_hbm.at[ebuf[bn].at[1]], rbuf[bn], gsem[bn]).start()
                pltpu.make_async_copy(
                    vdata_hbm.at[w, k + 1], vbuf[bn], vsem[bn]).start()
                pltpu.make_async_copy(
                    x_hbm.at[ebuf[b].at[1]], rbuf[b], gsem[b]).wait()
                pltpu.make_async_copy(
                    vdata_hbm.at[w, k], vbuf[b], vsem[b]).wait()
                scale(b, None)
                pltpu.async_copy(rbuf[b], acc.at[ebuf[b].at[0]], ssem[b],
                                 add=True)
            return carry

        lax.fori_loop(0, n_chunk // 2, pair_body, 0)
        pltpu.make_async_copy(rbuf[1], acc.at[ebuf[1].at[0]], ssem[1]).wait()
        pltpu.make_async_copy(x_hbm.at[ebuf[0].at[1]], rbuf[0], gsem[0]).wait()
        pltpu.make_async_copy(vdata_hbm.at[w, 0], vbuf[0], vsem[0]).wait()
        plsc.subcore_barrier()
        pltpu.sync_copy(acc.at[pl.ds(r0, r1)], out_hbm.at[cid, pl.ds(r0, r1)])

        @pl.when(sid == 0)
        def _out_tail():
            pltpu.sync_copy(acc.at[pl.ds(r1 * _NS, tail)],
                            out_hbm.at[cid, pl.ds(r1 * _NS, tail)])

    return spmm


def kernel(user_feature, edge_index, adj_values, gamma, W_des, b_des,
           W_tweet, b_tweet, W_num, b_num, W_cat, b_cat, W1, W2):
    n, f = user_feature.shape
    num = W_num.shape[0]
    cat = W_cat.shape[0]
    des = W_des.shape[0]
    q = W_des.shape[1]

    wb = jnp.zeros((f, 4 * q), jnp.float32)
    wb = wb.at[:num, :q].set(W_num)
    wb = wb.at[num:num + cat, q:2 * q].set(W_cat)
    wb = wb.at[num + cat:num + cat + des, 2 * q:3 * q].set(W_des)
    wb = wb.at[num + cat + des:, 3 * q:].set(W_tweet)
    bb = jnp.concatenate([b_num, b_cat, b_des, b_tweet]).reshape(1, -1)

    e = edge_index.shape[1]
    nw = _NC * _NS
    n_chunk = -(-e // (nw * _C))
    n_chunk += n_chunk % 2
    e_pad = n_chunk * _C * nw
    pad = e_pad - e
    rows = jnp.pad(edge_index[0].astype(jnp.int32), (0, pad))
    cols = jnp.pad(edge_index[1].astype(jnp.int32), (0, pad))
    edata = jnp.stack([rows, cols])
    edata = edata.reshape(2, nw, n_chunk, _C).transpose(1, 2, 0, 3)
    edata = jnp.pad(edata, ((0, 0), (0, 1), (0, 0), (0, 0)))
    vdata = jnp.pad(adj_values, (0, pad)).reshape(nw, n_chunk, _C)
    vdata = jnp.pad(vdata, ((0, 0), (0, 1), (0, 0)))

    hid = W1.shape[1]
    out_d = W2.shape[1]
    zeros_h = jnp.zeros((n // _NS, hid), jnp.float32)
    w2p = jnp.zeros((hid, hid), jnp.float32).at[:, :out_d].set(W2)
    gtp = jnp.zeros((hid, gamma.shape[0]), jnp.float32).at[:out_d].set(gamma.T)

    spmm = _make_spmm(n, hid, n_chunk)
    h = _encoder(user_feature, wb, bb, W1)
    p = spmm(h, edata, vdata, zeros_h)
    g = _mid(p, w2p)
    qp = spmm(g, edata, vdata, zeros_h)
    z_mean, clusters = _zclusters(qp, gtp, out_d)
    recon = _recon(z_mean)
    return (recon, clusters, z_mean)


# speedup vs baseline: 1.0003x; 1.0003x over previous
import functools

import jax
import jax.numpy as jnp
from jax import lax
from jax.experimental import pallas as pl
from jax.experimental.pallas import tpu as pltpu
from jax.experimental.pallas import tpu_sc as plsc

_NC = 2
_NS = 16
_C = 128


def _enc_body(uf_ref, wb_ref, b_ref, w1_ref, h_ref):
    u = jnp.dot(uf_ref[...], wb_ref[...], preferred_element_type=jnp.float32)
    u = u + b_ref[...]
    u = jnp.where(u > 0, u, 0.01 * u)
    h_ref[...] = jnp.dot(u, w1_ref[...], preferred_element_type=jnp.float32)


def _encoder(uf, wb, b, w1, bn=1000, interpret=False):
    n, f = uf.shape
    d = wb.shape[1]
    return pl.pallas_call(
        _enc_body,
        grid=(n // bn,),
        in_specs=[
            pl.BlockSpec((bn, f), lambda i: (i, 0)),
            pl.BlockSpec((f, d), lambda i: (0, 0)),
            pl.BlockSpec((1, d), lambda i: (0, 0)),
            pl.BlockSpec((d, d), lambda i: (0, 0)),
        ],
        out_specs=pl.BlockSpec((bn, d), lambda i: (i, 0)),
        out_shape=jax.ShapeDtypeStruct((n, d), jnp.float32),
        interpret=interpret,
    )(uf, wb, b, w1)


def _mid_body(p_ref, w2_ref, g_ref):
    z = p_ref[0] + p_ref[1]
    g_ref[...] = jnp.dot(z, w2_ref[...], preferred_element_type=jnp.float32)


def _mid(p, w2, bn=1000, interpret=False):
    _, n, d = p.shape
    o = w2.shape[1]
    return pl.pallas_call(
        _mid_body,
        grid=(n // bn,),
        in_specs=[
            pl.BlockSpec((2, bn, d), lambda i: (0, i, 0)),
            pl.BlockSpec((d, o), lambda i: (0, 0)),
        ],
        out_specs=pl.BlockSpec((bn, o), lambda i: (i, 0)),
        out_shape=jax.ShapeDtypeStruct((n, o), jnp.float32),
        interpret=interpret,
    )(p, w2)


def _zc_body(q_ref, gt_ref, z_ref, c_ref):
    zf = q_ref[0] + q_ref[1]
    z_ref[...] = zf[:, :z_ref.shape[1]]
    zn = jnp.sum(zf * zf, axis=1, keepdims=True)
    gt = gt_ref[...]
    gn = jnp.sum(gt * gt, axis=0, keepdims=True)
    cross = jnp.dot(zf, gt, preferred_element_type=jnp.float32)
    c_ref[...] = zn + gn - 2.0 * cross


def _zclusters(q, gt, out_d, bn=1000, interpret=False):
    _, n, dp = q.shape
    k = gt.shape[1]
    return pl.pallas_call(
        _zc_body,
        grid=(n // bn,),
        in_specs=[
            pl.BlockSpec((2, bn, dp), lambda i: (0, i, 0)),
            pl.BlockSpec((dp, k), lambda i: (0, 0)),
        ],
        out_specs=[
            pl.BlockSpec((bn, out_d), lambda i: (i, 0)),
            pl.BlockSpec((bn, k), lambda i: (i, 0)),
        ],
        out_shape=[
            jax.ShapeDtypeStruct((n, out_d), jnp.float32),
            jax.ShapeDtypeStruct((n, k), jnp.float32),
        ],
        interpret=interpret,
    )(q, gt)


def _recon_body(zi_ref, zj_ref, r_ref):
    r_ref[...] = lax.dot_general(
        zi_ref[...], zj_ref[...], (((1,), (1,)), ((), ())),
        preferred_element_type=jnp.float32)


def _recon(z, bn=400, interpret=False):
    n, d = z.shape
    return pl.pallas_call(
        _recon_body,
        grid=(n // bn,),
        in_specs=[
            pl.BlockSpec((bn, d), lambda i: (i, 0)),
            pl.BlockSpec((n, d), lambda i: (0, 0)),
        ],
        out_specs=pl.BlockSpec((bn, n), lambda i: (i, 0)),
        out_shape=jax.ShapeDtypeStruct((n, n), jnp.float32),
        interpret=interpret,
    )(z, z)


def _make_spmm(n_nodes, d, n_chunk):
    groups = d // 16
    r1 = (n_nodes // _NS) & ~7
    tail = n_nodes - r1 * _NS
    mesh = plsc.VectorSubcoreMesh(core_axis_name="c", subcore_axis_name="s")
    dnums = lax.GatherDimensionNumbers(
        offset_dims=(), collapsed_slice_dims=(0,), start_index_map=(0,))

    @functools.partial(
        pl.kernel, mesh=mesh,
        out_type=jax.ShapeDtypeStruct((_NC, n_nodes, d), jnp.float32),
        scratch_types=[
            pltpu.VMEM((2, _C), jnp.int32),
            pltpu.VMEM((2, _C), jnp.int32),
            pltpu.VMEM((_C,), jnp.float32),
            pltpu.VMEM((_C,), jnp.float32),
            pltpu.VMEM((_C, d), jnp.float32),
            pltpu.VMEM((_C, d), jnp.float32),
            pltpu.VMEM_SHARED((n_nodes, d), jnp.float32),
            pltpu.SemaphoreType.DMA,
            pltpu.SemaphoreType.DMA,
            pltpu.SemaphoreType.DMA,
            pltpu.SemaphoreType.DMA,
            pltpu.SemaphoreType.DMA,
            pltpu.SemaphoreType.DMA,
        ],
    )
    def spmm(x_hbm, edata_hbm, vdata_hbm, zeros_hbm, out_hbm,
             eb0, eb1, vb0, vb1, rb0, rb1, acc, gs0, gs1, ss0, ss1, vs0, vs1):
        ebuf, rbuf, gsem, ssem = (eb0, eb1), (rb0, rb1), (gs0, gs1), (ss0, ss1)
        vbuf, vsem = (vb0, vb1), (vs0, vs1)
        cid = lax.axis_index("c")
        sid = lax.axis_index("s")
        w = cid * _NS + sid
        r0 = sid * r1
        pltpu.sync_copy(zeros_hbm.at[pl.ds(0, r1)], acc.at[pl.ds(r0, r1)])

        @pl.when(sid == 0)
        def _zero_tail():
            pltpu.sync_copy(zeros_hbm.at[pl.ds(0, tail)],
                            acc.at[pl.ds(r1 * _NS, tail)])

        plsc.subcore_barrier()

        def scale(b, _unused):
            def scale_body(t, c2):
                vs = vbuf[b][pl.ds(t * 16, 16)]
                for e in range(16):
                    vv = lax.gather(
                        vs, jnp.full((16, 1), e, jnp.int32), dnums, (1,),
                        mode=lax.GatherScatterMode.PROMISE_IN_BOUNDS)
                    r = t * 16 + e
                    for g in range(groups):
                        sl = pl.ds(g * 16, 16)
                        rbuf[b][r, sl] = rbuf[b][r, sl] * vv
                return c2

            lax.fori_loop(0, _C // 16, scale_body, 0)

        vbase = w * (n_chunk * _C)
        pltpu.sync_copy(edata_hbm.at[w, 0], ebuf[0])
        pltpu.make_async_copy(x_hbm.at[ebuf[0].at[1]], rbuf[0], gsem[0]).start()
        pltpu.make_async_copy(vdata_hbm.at[pl.ds(vbase, _C)], vbuf[0],
                              vsem[0]).start()

        def pair_body(t, carry):
            for b in range(2):
                k = 2 * t + b
                bn = 1 - b

                @pl.when(k >= 1)
                def _wait_prev_scatter():
                    pltpu.make_async_copy(
                        rbuf[bn], acc.at[ebuf[bn].at[0]], ssem[bn]).wait()

                pltpu.sync_copy(edata_hbm.at[w, k + 1], ebuf[bn])
                pltpu.make_async_copy(
                    x_hbm.at[ebuf[bn].at[1]], rbuf[bn], gsem[bn]).start()
                pltpu.make_async_copy(
                    vdata_hbm.at[pl.ds(vbase + (k + 1) * _C, _C)], vbuf[bn],
                    vsem[bn]).start()
                pltpu.make_async_copy(
                    x_hbm.at[ebuf[b].at[1]], rbuf[b], gsem[b]).wait()
                pltpu.make_async_copy(
                    vdata_hbm.at[pl.ds(vbase + k * _C, _C)], vbuf[b],
                    vsem[b]).wait()
                scale(b, None)
                pltpu.async_copy(rbuf[b], acc.at[ebuf[b].at[0]], ssem[b],
                                 add=True)
            return carry

        lax.fori_loop(0, n_chunk // 2, pair_body, 0)
        pltpu.make_async_copy(rbuf[1], acc.at[ebuf[1].at[0]], ssem[1]).wait()
        pltpu.make_async_copy(x_hbm.at[ebuf[0].at[1]], rbuf[0], gsem[0]).wait()
        pltpu.make_async_copy(vdata_hbm.at[pl.ds(vbase, _C)], vbuf[0],
                              vsem[0]).wait()
        plsc.subcore_barrier()
        pltpu.sync_copy(acc.at[pl.ds(r0, r1)], out_hbm.at[cid, pl.ds(r0, r1)])

        @pl.when(sid == 0)
        def _out_tail():
            pltpu.sync_copy(acc.at[pl.ds(r1 * _NS, tail)],
                            out_hbm.at[cid, pl.ds(r1 * _NS, tail)])

    return spmm


def kernel(user_feature, edge_index, adj_values, gamma, W_des, b_des,
           W_tweet, b_tweet, W_num, b_num, W_cat, b_cat, W1, W2):
    n, f = user_feature.shape
    num = W_num.shape[0]
    cat = W_cat.shape[0]
    des = W_des.shape[0]
    q = W_des.shape[1]

    wb = jnp.zeros((f, 4 * q), jnp.float32)
    wb = wb.at[:num, :q].set(W_num)
    wb = wb.at[num:num + cat, q:2 * q].set(W_cat)
    wb = wb.at[num + cat:num + cat + des, 2 * q:3 * q].set(W_des)
    wb = wb.at[num + cat + des:, 3 * q:].set(W_tweet)
    bb = jnp.concatenate([b_num, b_cat, b_des, b_tweet]).reshape(1, -1)

    e = edge_index.shape[1]
    nw = _NC * _NS
    n_chunk = -(-e // (nw * _C))
    n_chunk += n_chunk % 2
    e_pad = n_chunk * _C * nw
    pad = e_pad - e
    rows = jnp.pad(edge_index[0].astype(jnp.int32), (0, pad))
    cols = jnp.pad(edge_index[1].astype(jnp.int32), (0, pad))
    edata = jnp.stack([rows, cols])
    edata = edata.reshape(2, nw, n_chunk, _C).transpose(1, 2, 0, 3)
    edata = jnp.pad(edata, ((0, 0), (0, 1), (0, 0), (0, 0)))
    vdata = jnp.pad(adj_values, (0, pad + _C))

    hid = W1.shape[1]
    out_d = W2.shape[1]
    zeros_h = jnp.zeros((n // _NS, hid), jnp.float32)
    w2p = jnp.zeros((hid, hid), jnp.float32).at[:, :out_d].set(W2)
    gtp = jnp.zeros((hid, gamma.shape[0]), jnp.float32).at[:out_d].set(gamma.T)

    spmm = _make_spmm(n, hid, n_chunk)
    h = _encoder(user_feature, wb, bb, W1)
    p = spmm(h, edata, vdata, zeros_h)
    g = _mid(p, w2p)
    qp = spmm(g, edata, vdata, zeros_h)
    z_mean, clusters = _zclusters(qp, gtp, out_d)
    recon = _recon(z_mean)
    return (recon, clusters, z_mean)


# speedup vs baseline: 1.2792x; 1.2789x over previous
import functools

import jax
import jax.numpy as jnp
from jax import lax
from jax.experimental import pallas as pl
from jax.experimental.pallas import tpu as pltpu
from jax.experimental.pallas import tpu_sc as plsc

_NC = 2
_NS = 16
_C = 128


def _enc_body(uf_ref, wb_ref, b_ref, w1_ref, h_ref):
    u = jnp.dot(uf_ref[...], wb_ref[...], preferred_element_type=jnp.float32)
    u = u + b_ref[...]
    u = jnp.where(u > 0, u, 0.01 * u)
    h_ref[...] = jnp.dot(u, w1_ref[...], preferred_element_type=jnp.float32)


def _encoder(uf, wb, b, w1, bn=1000, interpret=False):
    n, f = uf.shape
    d = wb.shape[1]
    return pl.pallas_call(
        _enc_body,
        grid=(n // bn,),
        in_specs=[
            pl.BlockSpec((bn, f), lambda i: (i, 0)),
            pl.BlockSpec((f, d), lambda i: (0, 0)),
            pl.BlockSpec((1, d), lambda i: (0, 0)),
            pl.BlockSpec((d, d), lambda i: (0, 0)),
        ],
        out_specs=pl.BlockSpec((bn, d), lambda i: (i, 0)),
        out_shape=jax.ShapeDtypeStruct((n, d), jnp.float32),
        interpret=interpret,
    )(uf, wb, b, w1)


def _mid_body(p_ref, w2_ref, g_ref):
    z = p_ref[0] + p_ref[1]
    g_ref[...] = jnp.dot(z, w2_ref[...], preferred_element_type=jnp.float32)


def _mid(p, w2, bn=1000, interpret=False):
    _, n, d = p.shape
    o = w2.shape[1]
    return pl.pallas_call(
        _mid_body,
        grid=(n // bn,),
        in_specs=[
            pl.BlockSpec((2, bn, d), lambda i: (0, i, 0)),
            pl.BlockSpec((d, o), lambda i: (0, 0)),
        ],
        out_specs=pl.BlockSpec((bn, o), lambda i: (i, 0)),
        out_shape=jax.ShapeDtypeStruct((n, o), jnp.float32),
        interpret=interpret,
    )(p, w2)


def _zc_body(q_ref, gt_ref, z_ref, c_ref):
    zf = q_ref[0] + q_ref[1]
    z_ref[...] = zf[:, :z_ref.shape[1]]
    zn = jnp.sum(zf * zf, axis=1, keepdims=True)
    gt = gt_ref[...]
    gn = jnp.sum(gt * gt, axis=0, keepdims=True)
    cross = jnp.dot(zf, gt, preferred_element_type=jnp.float32)
    c_ref[...] = zn + gn - 2.0 * cross


def _zclusters(q, gt, out_d, bn=1000, interpret=False):
    _, n, dp = q.shape
    k = gt.shape[1]
    return pl.pallas_call(
        _zc_body,
        grid=(n // bn,),
        in_specs=[
            pl.BlockSpec((2, bn, dp), lambda i: (0, i, 0)),
            pl.BlockSpec((dp, k), lambda i: (0, 0)),
        ],
        out_specs=[
            pl.BlockSpec((bn, out_d), lambda i: (i, 0)),
            pl.BlockSpec((bn, k), lambda i: (i, 0)),
        ],
        out_shape=[
            jax.ShapeDtypeStruct((n, out_d), jnp.float32),
            jax.ShapeDtypeStruct((n, k), jnp.float32),
        ],
        interpret=interpret,
    )(q, gt)


def _recon_body(zi_ref, zj_ref, r_ref):
    r_ref[...] = lax.dot_general(
        zi_ref[...], zj_ref[...], (((1,), (1,)), ((), ())),
        preferred_element_type=jnp.float32)


def _recon(z, bn=400, interpret=False):
    n, d = z.shape
    return pl.pallas_call(
        _recon_body,
        grid=(n // bn,),
        in_specs=[
            pl.BlockSpec((bn, d), lambda i: (i, 0)),
            pl.BlockSpec((n, d), lambda i: (0, 0)),
        ],
        out_specs=pl.BlockSpec((bn, n), lambda i: (i, 0)),
        out_shape=jax.ShapeDtypeStruct((n, n), jnp.float32),
        interpret=interpret,
    )(z, z)


def _make_spmm(n_nodes, d, n_chunk):
    groups = d // 16
    r1 = (n_nodes // _NS) & ~7
    tail = n_nodes - r1 * _NS
    mesh = plsc.VectorSubcoreMesh(core_axis_name="c", subcore_axis_name="s")
    dnums = lax.GatherDimensionNumbers(
        offset_dims=(), collapsed_slice_dims=(0,), start_index_map=(0,))

    @functools.partial(
        pl.kernel, mesh=mesh,
        out_type=jax.ShapeDtypeStruct((_NC, n_nodes, d), jnp.float32),
        scratch_types=[
            pltpu.VMEM((_C,), jnp.int32),
            pltpu.VMEM((_C,), jnp.int32),
            pltpu.VMEM((_C,), jnp.int32),
            pltpu.VMEM((_C,), jnp.int32),
            pltpu.VMEM((_C,), jnp.float32),
            pltpu.VMEM((_C,), jnp.float32),
            pltpu.VMEM((_C, d), jnp.float32),
            pltpu.VMEM((_C, d), jnp.float32),
            pltpu.VMEM_SHARED((n_nodes, d), jnp.float32),
        ] + [pltpu.SemaphoreType.DMA] * 8,
    )
    def spmm(x2_hbm, rows_hbm, cols_hbm, vals_hbm, zeros_hbm, out_hbm,
             ro0, ro1, co0, co1, vb0, vb1, rb0, rb1, acc,
             gs0, gs1, ss0, ss1, vs0, vs1, rs0, rs1):
        robuf, cobuf = (ro0, ro1), (co0, co1)
        rbuf, gsem, ssem = (rb0, rb1), (gs0, gs1), (ss0, ss1)
        vbuf, vsem, rsem = (vb0, vb1), (vs0, vs1), (rs0, rs1)
        cid = lax.axis_index("c")
        sid = lax.axis_index("s")
        w = cid * _NS + sid
        r0 = sid * r1
        pltpu.sync_copy(zeros_hbm.at[pl.ds(0, r1)], acc.at[pl.ds(r0, r1)])

        @pl.when(sid == 0)
        def _zero_tail():
            pltpu.sync_copy(zeros_hbm.at[pl.ds(0, tail)],
                            acc.at[pl.ds(r1 * _NS, tail)])

        plsc.subcore_barrier()

        def scale(b, _unused):
            def scale_body(t, c2):
                vs = vbuf[b][pl.ds(t * 16, 16)]
                for e in range(16):
                    vv = lax.gather(
                        vs, jnp.full((16, 1), e, jnp.int32), dnums, (1,),
                        mode=lax.GatherScatterMode.PROMISE_IN_BOUNDS)
                    r = t * 16 + e
                    for g in range(groups):
                        sl = pl.ds(g * 16, 16)
                        rbuf[b][r, sl] = rbuf[b][r, sl] * vv
                return c2

            lax.fori_loop(0, _C // 16, scale_body, 0)

        base = w * (n_chunk * _C)

        x_hbm = x2_hbm.at[cid]

        def fetch(slot, off):
            pltpu.sync_copy(cols_hbm.at[pl.ds(off, _C)], cobuf[slot])
            pltpu.make_async_copy(
                x_hbm.at[cobuf[slot]], rbuf[slot], gsem[slot]).start()
            pltpu.make_async_copy(
                rows_hbm.at[pl.ds(off, _C)], robuf[slot], rsem[slot]).start()
            pltpu.make_async_copy(
                vals_hbm.at[pl.ds(off, _C)], vbuf[slot], vsem[slot]).start()

        def drain_fetch(slot, off):
            pltpu.make_async_copy(
                x_hbm.at[cobuf[slot]], rbuf[slot], gsem[slot]).wait()
            pltpu.make_async_copy(
                rows_hbm.at[pl.ds(off, _C)], robuf[slot], rsem[slot]).wait()
            pltpu.make_async_copy(
                vals_hbm.at[pl.ds(off, _C)], vbuf[slot], vsem[slot]).wait()

        fetch(0, base)

        def pair_body(t, carry):
            for b in range(2):
                k = 2 * t + b
                bn = 1 - b

                @pl.when(k >= 1)
                def _wait_prev_scatter():
                    pltpu.make_async_copy(
                        rbuf[bn], acc.at[robuf[bn]], ssem[bn]).wait()

                fetch(bn, base + (k + 1) * _C)
                drain_fetch(b, base + k * _C)
                scale(b, None)
                pltpu.async_copy(rbuf[b], acc.at[robuf[b]], ssem[b], add=True)
            return carry

        lax.fori_loop(0, n_chunk // 2, pair_body, 0)
        pltpu.make_async_copy(rbuf[1], acc.at[robuf[1]], ssem[1]).wait()
        drain_fetch(0, base)
        plsc.subcore_barrier()
        pltpu.sync_copy(acc.at[pl.ds(r0, r1)], out_hbm.at[cid, pl.ds(r0, r1)])

        @pl.when(sid == 0)
        def _out_tail():
            pltpu.sync_copy(acc.at[pl.ds(r1 * _NS, tail)],
                            out_hbm.at[cid, pl.ds(r1 * _NS, tail)])

    return spmm


def kernel(user_feature, edge_index, adj_values, gamma, W_des, b_des,
           W_tweet, b_tweet, W_num, b_num, W_cat, b_cat, W1, W2):
    n, f = user_feature.shape
    num = W_num.shape[0]
    cat = W_cat.shape[0]
    des = W_des.shape[0]
    q = W_des.shape[1]

    wb = jnp.zeros((f, 4 * q), jnp.float32)
    wb = wb.at[:num, :q].set(W_num)
    wb = wb.at[num:num + cat, q:2 * q].set(W_cat)
    wb = wb.at[num + cat:num + cat + des, 2 * q:3 * q].set(W_des)
    wb = wb.at[num + cat + des:, 3 * q:].set(W_tweet)
    bb = jnp.concatenate([b_num, b_cat, b_des, b_tweet]).reshape(1, -1)

    e = edge_index.shape[1]
    nw = _NC * _NS
    n_chunk = -(-e // (nw * _C))
    n_chunk += n_chunk % 2
    e_pad = n_chunk * _C * nw
    pad = e_pad - e
    rows = jnp.pad(edge_index[0].astype(jnp.int32), (0, pad + _C))
    cols = jnp.pad(edge_index[1].astype(jnp.int32), (0, pad + _C))
    vals = jnp.pad(adj_values, (0, pad + _C))

    hid = W1.shape[1]
    out_d = W2.shape[1]
    zeros_h = jnp.zeros((n // _NS, hid), jnp.float32)
    w2p = jnp.zeros((hid, hid), jnp.float32).at[:, :out_d].set(W2)
    gtp = jnp.zeros((hid, gamma.shape[0]), jnp.float32).at[:out_d].set(gamma.T)
    spmm = _make_spmm(n, hid, n_chunk)
    h = _encoder(user_feature, wb, bb, W1)
    p = spmm(jnp.stack([h, h]), rows, cols, vals, zeros_h)
    g = _mid(p, w2p)
    qp = spmm(jnp.stack([g, g]), rows, cols, vals, zeros_h)
    z_mean, clusters = _zclusters(qp, gtp, out_d)
    recon = _recon(z_mean)
    return (recon, clusters, z_mean)
